# baseline (speedup 1.0000x reference)
import functools

import jax
import jax.numpy as jnp
from jax import lax
from jax.experimental import pallas as pl
from jax.experimental.pallas import tpu as pltpu
from jax.experimental.pallas import tpu_sc as plsc

NE = 160000
NT = 320000
F = 128
NB = 8
NRBF = 16
NSBF = 42


BLK_E = 8000
BLK_T = 8000


def _swish(v):
    return v * jax.nn.sigmoid(v)


def _dot(a, b):
    return jnp.dot(a, b, preferred_element_type=jnp.float32)


def _pre_body(x_ref, rbf_ref, wji, bji, wkj, bkj, wrbf, xji_ref, h_ref):
    xb = x_ref[...].astype(jnp.bfloat16)
    xji_ref[...] = _swish(_dot(xb, wji[...]) + bji[...])
    g = _dot(rbf_ref[...].astype(jnp.bfloat16), wrbf[...])
    h_ref[...] = _swish(_dot(xb, wkj[...]) + bkj[...]) * g


def _pre_call(x, rbf, W_ji, b_ji, W_kj, b_kj, W_rbf):
    n = x.shape[0]
    grid = (n // BLK_E,)
    full = lambda shape: pl.BlockSpec(shape, lambda i: (0,) * len(shape))
    row = lambda w: pl.BlockSpec((BLK_E, w), lambda i: (i, 0))
    return pl.pallas_call(
        _pre_body,
        grid=grid,
        in_specs=[row(F), row(NRBF), full((F, F)), full((1, F)),
                  full((F, F)), full((1, F)), full((NRBF, F))],
        out_specs=[row(F), row(F)],
        out_shape=[jax.ShapeDtypeStruct((n, F), jnp.float32),
                   jax.ShapeDtypeStruct((n, F), jnp.float32)],
    )(x, rbf, W_ji, b_ji.reshape(1, F), W_kj, b_kj.reshape(1, F), W_rbf)


def _bilin_body(g_ref, sbf_ref, wsbf, w2, m_ref):
    sp = _dot(sbf_ref[...], wsbf[...]).astype(jnp.bfloat16)
    gb = g_ref[...].astype(jnp.bfloat16)
    kb = jnp.concatenate([gb * sp[:, l:l + 1] for l in range(NB)], axis=1)
    m_ref[...] = _dot(kb, w2[...])


def _bilin_call(G, sbf, W_sbf, W2):
    grid = (NT // BLK_T,)
    full = lambda shape: pl.BlockSpec(shape, lambda i: (0,) * len(shape))
    row = lambda w: pl.BlockSpec((BLK_T, w), lambda i: (i, 0))
    return pl.pallas_call(
        _bilin_body,
        grid=grid,
        in_specs=[row(F), row(NSBF), full((NSBF, NB)), full((NB * F, F))],
        out_specs=row(F),
        out_shape=jax.ShapeDtypeStruct((NT, F), jnp.float32),
    )(G, sbf, W_sbf, W2)


def _post_body(*refs):
    x_ref, xji_ref, agg_ref = refs[0], refs[1], refs[2]
    ws = refs[3:-1]
    out_ref = refs[-1]
    nb = (len(ws) - 2 - 8) // 4
    k = 0
    bf = jnp.bfloat16
    x2 = xji_ref[...] + agg_ref[...]
    for _ in range(nb):
        hh = _swish(_dot(x2.astype(bf), ws[k][...]) + ws[k + 1][...])
        hh = _swish(_dot(hh.astype(bf), ws[k + 2][...]) + ws[k + 3][...])
        x2 = x2 + hh
        k += 4
    x2 = _swish(_dot(x2.astype(bf), ws[k][...]) + ws[k + 1][...])
    k += 2
    xx = x_ref[...] + x2
    while k < len(ws):
        hh = _swish(_dot(xx.astype(bf), ws[k][...]) + ws[k + 1][...])
        hh = _swish(_dot(hh.astype(bf), ws[k + 2][...]) + ws[k + 3][...])
        xx = xx + hh
        k += 4
    out_ref[...] = xx


def _post_call(x, xji, agg_padded, wlist):
    grid = (NE // BLK_E,)
    full = lambda shape: pl.BlockSpec(shape, lambda i: (0,) * len(shape))
    row = pl.BlockSpec((BLK_E, F), lambda i: (i, 0))
    return pl.pallas_call(
        _post_body,
        grid=grid,
        in_specs=[row, row, row] + [full(w.shape) for w in wlist],
        out_specs=row,
        out_shape=jax.ShapeDtypeStruct((NE, F), jnp.float32),
    )(x, xji, agg_padded, *wlist)


_NC = 2
_NS = 16
_NW = _NC * _NS
_TPW = NT // _NW
_GCH = 80
_NGCH = _TPW // _GCH

_SC_MESH = plsc.VectorSubcoreMesh(core_axis_name="c", subcore_axis_name="s")


@functools.partial(
    pl.kernel,
    out_type=jax.ShapeDtypeStruct((NT, F), jnp.float32),
    mesh=_SC_MESH,
    compiler_params=pltpu.CompilerParams(needs_layout_passes=False),
    scratch_types=[
        pltpu.VMEM((_TPW,), jnp.int32),
        pltpu.VMEM((_GCH, F), jnp.float32),
        pltpu.VMEM((_GCH, F), jnp.float32),
        pltpu.SemaphoreType.DMA,
        pltpu.SemaphoreType.DMA,
    ],
)
def _gather_k(h_hbm, idx_hbm, out_hbm, idx_v, buf0, buf1, sem0, sem1):
    wid = lax.axis_index("s") * _NC + lax.axis_index("c")
    base = wid * _TPW
    pltpu.sync_copy(idx_hbm.at[pl.ds(pl.multiple_of(base, 8), _TPW)], idx_v)

    def start(i, buf, sem):
        off = pl.multiple_of(i * _GCH, 8)
        pltpu.async_copy(h_hbm.at[idx_v.at[pl.ds(off, _GCH)]], buf, sem)

    def wait(buf, sem):
        pltpu.make_async_copy(h_hbm.at[idx_v.at[pl.ds(0, _GCH)]], buf, sem).wait()

    def wb(i, buf):
        off = pl.multiple_of(i * _GCH, 8)
        pltpu.sync_copy(buf, out_hbm.at[pl.ds(pl.multiple_of(base + off, 8), _GCH)])

    start(0, buf0, sem0)

    def pair(i2, c):
        i = i2 * 2
        start(i + 1, buf1, sem1)
        wait(buf0, sem0)
        wb(i, buf0)
        start(i + 2, buf0, sem0)
        wait(buf1, sem1)
        wb(i + 1, buf1)
        return c

    lax.fori_loop(0, (_NGCH - 1) // 2, pair, 0)
    wait(buf0, sem0)
    wb(_NGCH - 1, buf0)


_NPASS = 10
_RPB = 8448
_ACC_R = _RPB + 128
_NEPAD = _NPASS * _NC * _RPB
_ZB = 64
_SCH = 128
_LCAP_CK = (_TPW + _SCH - 1) // _SCH
_ZR = _ACC_R // _NS
_DR = _RPB // _NS


@functools.partial(
    pl.kernel,
    out_type=jax.ShapeDtypeStruct((_NEPAD, F), jnp.float32),
    mesh=_SC_MESH,
    compiler_params=pltpu.CompilerParams(needs_layout_passes=False),
    scratch_types=[
        pltpu.VMEM_SHARED((_ACC_R, F), jnp.float32),
        pltpu.VMEM((_TPW,), jnp.int32),
        pltpu.VMEM((_LCAP_CK * _SCH,), jnp.int32),
        pltpu.VMEM((2, _SCH), jnp.int32),
        pltpu.VMEM((_SCH, F), jnp.float32),
        pltpu.VMEM((_SCH, F), jnp.float32),
        pltpu.VMEM((_ZB, F), jnp.float32),
        pltpu.SemaphoreType.DMA,
        pltpu.SemaphoreType.DMA,
    ],
)
def _scatter_k(m_hbm, idr_hbm, out_hbm, acc, idr_v, wlist, didx, mbuf0, mbuf1,
               zbuf, sem0, sem1):
    cid = lax.axis_index("c")
    sid = lax.axis_index("s")
    wid = sid * _NC + cid
    base = wid * _TPW
    iota16 = lax.iota(jnp.int32, 16)
    zv = jnp.zeros((16,), jnp.float32)

    pltpu.sync_copy(idr_hbm.at[pl.ds(pl.multiple_of(base, 8), _TPW)], idr_v)

    def zb(t, c):
        zbuf[t // 8, pl.ds((t % 8) * 16, 16)] = zv
        return c
    lax.fori_loop(0, _ZB * 8, zb, 0)

    bv = jnp.zeros((16,), jnp.int32) + base

    def wf(t, c):
        wlist[pl.ds(t * 16, 16)] = bv
        return c
    lax.fori_loop(0, (_LCAP_CK * _SCH) // 16, wf, 0)

    def do_pass(p, c):
        b = p * _NC + cid
        lob = b * _RPB
        for j in range(_ZR // _ZB):
            pltpu.sync_copy(zbuf, acc.at[pl.ds(pl.multiple_of(sid * _ZR + j * _ZB, 8), _ZB)])
        rem = _ZR % _ZB
        if rem:
            pltpu.sync_copy(zbuf.at[pl.ds(0, rem)],
                            acc.at[pl.ds(pl.multiple_of(sid * _ZR + (_ZR // _ZB) * _ZB, 8), rem)])
        plsc.subcore_barrier()

        def scan(v, cnt):
            d0 = idr_v[pl.ds(v * 32, 16)]
            d1 = idr_v[pl.ds(v * 32 + 16, 16)]
            m0 = (d0 >= lob) & (d0 < lob + _RPB)
            m1 = (d1 >= lob) & (d1 < lob + _RPB)
            cs0 = plsc.cumsum(m0.astype(jnp.int32))
            cs1 = plsc.cumsum(m1.astype(jnp.int32))
            plsc.store_scatter(wlist, [cnt + cs0 - 1],
                               base + v * 32 + iota16, mask=m0)
            c0 = cnt + cs0[15]
            plsc.store_scatter(wlist, [c0 + cs1 - 1],
                               base + v * 32 + 16 + iota16, mask=m1)
            return c0 + cs1[15]
        cnt = lax.fori_loop(0, _TPW // 32, scan, 0)

        nck = (cnt + (_SCH - 1)) // _SCH
        bufs = (mbuf0, mbuf1)
        sems = (sem0, sem1)

        def start(kk, slot):
            off = pl.multiple_of(kk * _SCH, 8)
            pltpu.async_copy(m_hbm.at[wlist.at[pl.ds(off, _SCH)]],
                             bufs[slot], sems[slot])
            for j in range(_SCH // 16):
                wv = wlist[pl.ds(off + j * 16, 16)]
                dv = plsc.load_gather(idr_v, [wv - base]) - lob
                dv = jnp.where(off + j * 16 + iota16 < cnt, dv, _RPB)
                didx[slot, pl.ds(j * 16, 16)] = dv

        def finish(slot):
            pltpu.make_async_copy(m_hbm.at[wlist.at[pl.ds(0, _SCH)]],
                                  bufs[slot], sems[slot]).wait()
            pltpu.sync_copy(bufs[slot], acc.at[didx.at[slot]], add=True)

        @pl.when(nck > 0)
        def _():
            start(0, 0)

        def pair(k2, c2):
            k = k2 * 2

            @pl.when(k + 1 < nck)
            def _():
                start(k + 1, 1)
            finish(0)

            @pl.when(k + 2 < nck)
            def _():
                start(k + 2, 0)

            @pl.when(k + 1 < nck)
            def _():
                finish(1)
            return c2
        lax.fori_loop(0, (nck + 1) // 2, pair, 0)
        plsc.subcore_barrier()

        for j in range(_DR // 128):
            pltpu.sync_copy(acc.at[pl.ds(pl.multiple_of(sid * _DR + j * 128, 8), 128)],
                            out_hbm.at[pl.ds(pl.multiple_of(lob + sid * _DR + j * 128, 8), 128)])
        rem = _DR % 128
        if rem:
            pltpu.sync_copy(acc.at[pl.ds(pl.multiple_of(sid * _DR + (_DR // 128) * 128, 8), rem)],
                            out_hbm.at[pl.ds(pl.multiple_of(lob + sid * _DR + (_DR // 128) * 128, 8), rem)])
        plsc.subcore_barrier()
        return c

    lax.fori_loop(0, _NPASS, do_pass, 0)


def kernel(x, rbf, sbf, id_expand_kj, id_reduce_ji, W_rbf, W_sbf, W_ji, b_ji,
           W_kj, b_kj, W_bilin, W_before, b_before, W_final, b_final,
           W_after, b_after):
    bf = jnp.bfloat16
    W2 = W_bilin.transpose(2, 0, 1).reshape(NB * F, F).astype(bf)

    xji, h = _pre_call(x, rbf, W_ji.astype(bf), b_ji, W_kj.astype(bf),
                       b_kj, W_rbf.astype(bf))
    G = _gather_k(h, id_expand_kj)
    m = _bilin_call(G, sbf, W_sbf, W2)
    agg = _scatter_k(m, id_reduce_ji)

    wlist = []
    for i in range(W_before.shape[0]):
        wlist += [W_before[i, 0].astype(bf), b_before[i, 0].reshape(1, F),
                  W_before[i, 1].astype(bf), b_before[i, 1].reshape(1, F)]
    wlist += [W_final.astype(bf), b_final.reshape(1, F)]
    for i in range(W_after.shape[0]):
        wlist += [W_after[i, 0].astype(bf), b_after[i, 0].reshape(1, F),
                  W_after[i, 1].astype(bf), b_after[i, 1].reshape(1, F)]
    return _post_call(x, xji, agg, wlist)


# speedup vs baseline: 1.0014x; 1.0014x over previous
import functools

import jax
import jax.numpy as jnp
from jax import lax
from jax.experimental import pallas as pl
from jax.experimental.pallas import tpu as pltpu
from jax.experimental.pallas import tpu_sc as plsc

NE = 160000
NT = 320000
F = 128
NB = 8
NRBF = 16
NSBF = 42


BLK_E = 8000
BLK_T = 8000


def _swish(v):
    return v * jax.nn.sigmoid(v)


def _dot(a, b):
    return jnp.dot(a, b, preferred_element_type=jnp.float32)


def _pre_body(x_ref, rbf_ref, wji, bji, wkj, bkj, wrbf, xji_ref, h_ref):
    xb = x_ref[...].astype(jnp.bfloat16)
    xji_ref[...] = _swish(_dot(xb, wji[...]) + bji[...])
    g = _dot(rbf_ref[...].astype(jnp.bfloat16), wrbf[...])
    h_ref[...] = _swish(_dot(xb, wkj[...]) + bkj[...]) * g


def _pre_call(x, rbf, W_ji, b_ji, W_kj, b_kj, W_rbf):
    n = x.shape[0]
    grid = (n // BLK_E,)
    full = lambda shape: pl.BlockSpec(shape, lambda i: (0,) * len(shape))
    row = lambda w: pl.BlockSpec((BLK_E, w), lambda i: (i, 0))
    return pl.pallas_call(
        _pre_body,
        grid=grid,
        in_specs=[row(F), row(NRBF), full((F, F)), full((1, F)),
                  full((F, F)), full((1, F)), full((NRBF, F))],
        out_specs=[row(F), row(F)],
        out_shape=[jax.ShapeDtypeStruct((n, F), jnp.float32),
                   jax.ShapeDtypeStruct((n, F), jnp.float32)],
    )(x, rbf, W_ji, b_ji.reshape(1, F), W_kj, b_kj.reshape(1, F), W_rbf)


def _bilin_body(g_ref, sbf_ref, wsbf, w2, m_ref):
    sp = _dot(sbf_ref[...], wsbf[...]).astype(jnp.bfloat16)
    gb = g_ref[...].astype(jnp.bfloat16)
    kb = jnp.concatenate([gb * sp[:, l:l + 1] for l in range(NB)], axis=1)
    m_ref[...] = _dot(kb, w2[...])


def _bilin_call(G, sbf, W_sbf, W2):
    grid = (NT // BLK_T,)
    full = lambda shape: pl.BlockSpec(shape, lambda i: (0,) * len(shape))
    row = lambda w: pl.BlockSpec((BLK_T, w), lambda i: (i, 0))
    return pl.pallas_call(
        _bilin_body,
        grid=grid,
        in_specs=[row(F), row(NSBF), full((NSBF, NB)), full((NB * F, F))],
        out_specs=row(F),
        out_shape=jax.ShapeDtypeStruct((NT, F), jnp.float32),
    )(G, sbf, W_sbf, W2)


def _post_body(*refs):
    x_ref, xji_ref, agg_ref = refs[0], refs[1], refs[2]
    ws = refs[3:-1]
    out_ref = refs[-1]
    nb = (len(ws) - 2 - 8) // 4
    k = 0
    bf = jnp.bfloat16
    x2 = xji_ref[...] + agg_ref[...]
    for _ in range(nb):
        hh = _swish(_dot(x2.astype(bf), ws[k][...]) + ws[k + 1][...])
        hh = _swish(_dot(hh.astype(bf), ws[k + 2][...]) + ws[k + 3][...])
        x2 = x2 + hh
        k += 4
    x2 = _swish(_dot(x2.astype(bf), ws[k][...]) + ws[k + 1][...])
    k += 2
    xx = x_ref[...] + x2
    while k < len(ws):
        hh = _swish(_dot(xx.astype(bf), ws[k][...]) + ws[k + 1][...])
        hh = _swish(_dot(hh.astype(bf), ws[k + 2][...]) + ws[k + 3][...])
        xx = xx + hh
        k += 4
    out_ref[...] = xx


def _post_call(x, xji, agg_padded, wlist):
    grid = (NE // BLK_E,)
    full = lambda shape: pl.BlockSpec(shape, lambda i: (0,) * len(shape))
    row = pl.BlockSpec((BLK_E, F), lambda i: (i, 0))
    return pl.pallas_call(
        _post_body,
        grid=grid,
        in_specs=[row, row, row] + [full(w.shape) for w in wlist],
        out_specs=row,
        out_shape=jax.ShapeDtypeStruct((NE, F), jnp.float32),
    )(x, xji, agg_padded, *wlist)


_NC = 2
_NS = 16
_NW = _NC * _NS
_TPW = NT // _NW
_GCH = 80
_NGCH = _TPW // _GCH

_SC_MESH = plsc.VectorSubcoreMesh(core_axis_name="c", subcore_axis_name="s")


@functools.partial(
    pl.kernel,
    out_type=jax.ShapeDtypeStruct((NT, F), jnp.float32),
    mesh=_SC_MESH,
    compiler_params=pltpu.CompilerParams(needs_layout_passes=False),
    scratch_types=[
        pltpu.VMEM((_TPW,), jnp.int32),
        pltpu.VMEM((_GCH, F), jnp.float32),
        pltpu.VMEM((_GCH, F), jnp.float32),
        pltpu.SemaphoreType.DMA,
        pltpu.SemaphoreType.DMA,
    ],
)
def _gather_k(h_hbm, idx_hbm, out_hbm, idx_v, buf0, buf1, sem0, sem1):
    wid = lax.axis_index("s") * _NC + lax.axis_index("c")
    base = wid * _TPW
    pltpu.sync_copy(idx_hbm.at[pl.ds(pl.multiple_of(base, 8), _TPW)], idx_v)

    def start(i, buf, sem):
        off = pl.multiple_of(i * _GCH, 8)
        pltpu.async_copy(h_hbm.at[idx_v.at[pl.ds(off, _GCH)]], buf, sem)

    def wait(buf, sem):
        pltpu.make_async_copy(h_hbm.at[idx_v.at[pl.ds(0, _GCH)]], buf, sem).wait()

    def wb(i, buf):
        off = pl.multiple_of(i * _GCH, 8)
        pltpu.sync_copy(buf, out_hbm.at[pl.ds(pl.multiple_of(base + off, 8), _GCH)])

    start(0, buf0, sem0)

    def pair(i2, c):
        i = i2 * 2
        start(i + 1, buf1, sem1)
        wait(buf0, sem0)
        wb(i, buf0)
        start(i + 2, buf0, sem0)
        wait(buf1, sem1)
        wb(i + 1, buf1)
        return c

    lax.fori_loop(0, (_NGCH - 1) // 2, pair, 0)
    wait(buf0, sem0)
    wb(_NGCH - 1, buf0)


_NPASS = 10
_RPB = 8448
_ACC_R = _RPB + 128
_NEPAD = _NPASS * _NC * _RPB
_ZB = 64
_SCH = 128
_LCAP_CK = (_TPW + _SCH - 1) // _SCH
_ZR = _ACC_R // _NS
_DR = _RPB // _NS


@functools.partial(
    pl.kernel,
    out_type=jax.ShapeDtypeStruct((_NEPAD, F), jnp.float32),
    mesh=_SC_MESH,
    compiler_params=pltpu.CompilerParams(needs_layout_passes=False),
    scratch_types=[
        pltpu.VMEM_SHARED((_ACC_R, F), jnp.float32),
        pltpu.VMEM((_TPW,), jnp.int32),
        pltpu.VMEM((_LCAP_CK * _SCH,), jnp.int32),
        pltpu.VMEM((2, _SCH), jnp.int32),
        pltpu.VMEM((_SCH, F), jnp.float32),
        pltpu.VMEM((_SCH, F), jnp.float32),
        pltpu.VMEM((_ZB, F), jnp.float32),
        pltpu.SemaphoreType.DMA,
        pltpu.SemaphoreType.DMA,
    ],
)
def _scatter_k(m_hbm, idr_hbm, out_hbm, acc, idr_v, wlist, didx, mbuf0, mbuf1,
               zbuf, sem0, sem1):
    cid = lax.axis_index("c")
    sid = lax.axis_index("s")
    wid = sid * _NC + cid
    base = wid * _TPW
    iota16 = lax.iota(jnp.int32, 16)
    zv = jnp.zeros((16,), jnp.float32)

    pltpu.sync_copy(idr_hbm.at[pl.ds(pl.multiple_of(base, 8), _TPW)], idr_v)

    def zb(t, c):
        zbuf[t // 8, pl.ds((t % 8) * 16, 16)] = zv
        return c
    lax.fori_loop(0, _ZB * 8, zb, 0)

    bv = jnp.zeros((16,), jnp.int32) + base

    def wf(t, c):
        wlist[pl.ds(t * 16, 16)] = bv
        return c
    lax.fori_loop(0, (_LCAP_CK * _SCH) // 16, wf, 0)

    def do_pass(p, c):
        b = p * _NC + cid
        lob = b * _RPB
        for j in range(_ZR // _ZB):
            pltpu.sync_copy(zbuf, acc.at[pl.ds(pl.multiple_of(sid * _ZR + j * _ZB, 8), _ZB)])
        rem = _ZR % _ZB
        if rem:
            pltpu.sync_copy(zbuf.at[pl.ds(0, rem)],
                            acc.at[pl.ds(pl.multiple_of(sid * _ZR + (_ZR // _ZB) * _ZB, 8), rem)])
        plsc.subcore_barrier()

        def scan(v, cnt):
            d = idr_v[pl.ds(v * 16, 16)]
            msk = (d >= lob) & (d < lob + _RPB)
            cs = plsc.cumsum(msk.astype(jnp.int32))
            plsc.store_scatter(wlist, [cnt + cs - 1], base + v * 16 + iota16,
                               mask=msk)
            return cnt + cs[15]
        cnt = lax.fori_loop(0, _TPW // 16, scan, 0)

        nck = (cnt + (_SCH - 1)) // _SCH
        bufs = (mbuf0, mbuf1)
        sems = (sem0, sem1)

        def start(kk, slot):
            off = pl.multiple_of(kk * _SCH, 8)
            pltpu.async_copy(m_hbm.at[wlist.at[pl.ds(off, _SCH)]],
                             bufs[slot], sems[slot])
            for j in range(_SCH // 16):
                wv = wlist[pl.ds(off + j * 16, 16)]
                dv = plsc.load_gather(idr_v, [wv - base]) - lob
                dv = jnp.where(off + j * 16 + iota16 < cnt, dv, _RPB)
                didx[slot, pl.ds(j * 16, 16)] = dv

        def finish(slot):
            pltpu.make_async_copy(m_hbm.at[wlist.at[pl.ds(0, _SCH)]],
                                  bufs[slot], sems[slot]).wait()
            pltpu.sync_copy(bufs[slot], acc.at[didx.at[slot]], add=True)

        @pl.when(nck > 0)
        def _():
            start(0, 0)

        def pair(k2, c2):
            k = k2 * 2

            @pl.when(k + 1 < nck)
            def _():
                start(k + 1, 1)
            finish(0)

            @pl.when(k + 2 < nck)
            def _():
                start(k + 2, 0)

            @pl.when(k + 1 < nck)
            def _():
                finish(1)
            return c2
        lax.fori_loop(0, (nck + 1) // 2, pair, 0)
        plsc.subcore_barrier()

        for j in range(_DR // 128):
            pltpu.sync_copy(acc.at[pl.ds(pl.multiple_of(sid * _DR + j * 128, 8), 128)],
                            out_hbm.at[pl.ds(pl.multiple_of(lob + sid * _DR + j * 128, 8), 128)])
        rem = _DR % 128
        if rem:
            pltpu.sync_copy(acc.at[pl.ds(pl.multiple_of(sid * _DR + (_DR // 128) * 128, 8), rem)],
                            out_hbm.at[pl.ds(pl.multiple_of(lob + sid * _DR + (_DR // 128) * 128, 8), rem)])
        plsc.subcore_barrier()
        return c

    lax.fori_loop(0, _NPASS, do_pass, 0)


def kernel(x, rbf, sbf, id_expand_kj, id_reduce_ji, W_rbf, W_sbf, W_ji, b_ji,
           W_kj, b_kj, W_bilin, W_before, b_before, W_final, b_final,
           W_after, b_after):
    bf = jnp.bfloat16
    W2 = W_bilin.transpose(2, 0, 1).reshape(NB * F, F).astype(bf)

    xji, h = _pre_call(x, rbf, W_ji.astype(bf), b_ji, W_kj.astype(bf),
                       b_kj, W_rbf.astype(bf))
    G = _gather_k(h, id_expand_kj)
    m = _bilin_call(G, sbf, W_sbf, W2)
    agg = _scatter_k(m, id_reduce_ji)

    wlist = []
    for i in range(W_before.shape[0]):
        wlist += [W_before[i, 0].astype(bf), b_before[i, 0].reshape(1, F),
                  W_before[i, 1].astype(bf), b_before[i, 1].reshape(1, F)]
    wlist += [W_final.astype(bf), b_final.reshape(1, F)]
    for i in range(W_after.shape[0]):
        wlist += [W_after[i, 0].astype(bf), b_after[i, 0].reshape(1, F),
                  W_after[i, 1].astype(bf), b_after[i, 1].reshape(1, F)]
    return _post_call(x, xji, agg, wlist)
